# baseline (speedup 1.0000x reference)
import functools

import jax
import jax.numpy as jnp
from jax import lax
from jax.experimental import pallas as pl
from jax.experimental.pallas import tpu as pltpu
from jax.experimental.pallas import tpu_sc as plsc

N = 10000
E = 320000
D_IN = 128
D_H = 64
W128 = 128

NC = 2
NS = 16
NW = NC * NS
CH = 128
CPW = 80
EPW = CH * CPW
E_PAD = NW * EPW
A = 10240
STRIPE = A // NS

_MESH = plsc.VectorSubcoreMesh(core_axis_name="c", subcore_axis_name="s")


@functools.partial(
    pl.kernel,
    out_type=jax.ShapeDtypeStruct((NC, A), jnp.float32),
    mesh=_MESH,
    compiler_params=pltpu.CompilerParams(needs_layout_passes=False),
    scratch_types=[
        pltpu.VMEM((CPW, CH), jnp.int32),
        pltpu.VMEM((A,), jnp.float32),
        pltpu.VMEM((NS, STRIPE), jnp.float32),
        pltpu.VMEM((STRIPE,), jnp.float32),
        pltpu.VMEM_SHARED((NS, A), jnp.float32),
        pltpu.SemaphoreType.DMA,
    ],
)
def _sc_degree(dst_hbm, zeros_hbm, out_hbm, di_all, hist, red, outv, hist_sh, sem):
    c = lax.axis_index("c")
    s = lax.axis_index("s")
    wid = c * NS + s
    pltpu.sync_copy(dst_hbm.at[wid], di_all)
    pltpu.sync_copy(zeros_hbm, hist)
    ones16 = jnp.full((16,), 1.0, jnp.float32)

    def body(j, carry):
        for t in range(CH // 16):
            idx = di_all[j, pl.ds(t * 16, 16)]
            plsc.addupdate_scatter(hist, [idx], ones16)
        return carry

    lax.fori_loop(0, CPW, body, 0)
    pltpu.sync_copy(hist, hist_sh.at[s])
    plsc.subcore_barrier()
    pltpu.sync_copy(hist_sh.at[:, pl.ds(s * STRIPE, STRIPE)], red)

    def rbody(t, carry):
        acc = jnp.zeros((16,), jnp.float32)
        for r in range(NS):
            acc = acc + red[r, pl.ds(t * 16, 16)]
        outv[pl.ds(t * 16, 16)] = acc
        return carry

    lax.fori_loop(0, STRIPE // 16, rbody, 0)
    pltpu.sync_copy(outv, out_hbm.at[c, pl.ds(s * STRIPE, STRIPE)])


NBUF = 8


@functools.partial(
    pl.kernel,
    out_type=jax.ShapeDtypeStruct((NC, A, D_H), jnp.float32),
    mesh=_MESH,
    compiler_params=pltpu.CompilerParams(use_tc_tiling_on_sc=False),
    scratch_types=[
        pltpu.VMEM((CPW, CH), jnp.int32),
        pltpu.VMEM((CPW, CH), jnp.int32),
        pltpu.VMEM((CH, D_H), jnp.float32),
        pltpu.VMEM((CH, D_H), jnp.float32),
        pltpu.VMEM((CH, D_H), jnp.float32),
        pltpu.VMEM((CH, D_H), jnp.float32),
        pltpu.VMEM((CH, D_H), jnp.float32),
        pltpu.VMEM((CH, D_H), jnp.float32),
        pltpu.VMEM((CH, D_H), jnp.float32),
        pltpu.VMEM((CH, D_H), jnp.float32),
        pltpu.VMEM_SHARED((A, D_H), jnp.float32),
        pltpu.SemaphoreType.DMA,
        pltpu.SemaphoreType.DMA,
        pltpu.SemaphoreType.DMA,
        pltpu.SemaphoreType.DMA,
        pltpu.SemaphoreType.DMA,
        pltpu.SemaphoreType.DMA,
        pltpu.SemaphoreType.DMA,
        pltpu.SemaphoreType.DMA,
        pltpu.SemaphoreType.DMA,
    ],
)
def _sc_edge_pass(y_hbm, src_hbm, dst_hbm, zeros_hbm, out_hbm,
                  si_all, di_all, rows0, rows1, rows2, rows3,
                  rows4, rows5, rows6, rows7,
                  acc_sh, gsem0, gsem1, gsem2, gsem3,
                  gsem4, gsem5, gsem6, gsem7, ssem):
    rows_bufs = (rows0, rows1, rows2, rows3, rows4, rows5, rows6, rows7)
    gsems = (gsem0, gsem1, gsem2, gsem3, gsem4, gsem5, gsem6, gsem7)
    c = lax.axis_index("c")
    s = lax.axis_index("s")
    wid = c * NS + s
    pltpu.sync_copy(src_hbm.at[wid], si_all)
    pltpu.sync_copy(dst_hbm.at[wid], di_all)
    pltpu.sync_copy(zeros_hbm, acc_sh.at[pl.ds(s * STRIPE, STRIPE)])
    plsc.subcore_barrier()

    def body(m, carry):
        gds, sds = [], []
        for b in range(NBUF):
            gds.append(pltpu.async_copy(
                y_hbm.at[si_all.at[m * NBUF + b]], rows_bufs[b], gsems[b]))
        for b in range(NBUF):
            gds[b].wait()
            sds.append(pltpu.async_copy(
                rows_bufs[b], acc_sh.at[di_all.at[m * NBUF + b]], ssem,
                add=True))
        for b in range(NBUF):
            sds[b].wait()
        return carry

    lax.fori_loop(0, CPW // NBUF, body, 0)
    plsc.subcore_barrier()
    pltpu.sync_copy(acc_sh.at[pl.ds(s * STRIPE, STRIPE)],
                    out_hbm.at[c, pl.ds(s * STRIPE, STRIPE)])


_R = 1000


def _tc_lin1_body(x_ref, w_ref, d0_ref, d1_ref, y_ref, dinv_ref):
    deg = d0_ref[...] + d1_ref[...] + 1.0
    dinv = lax.rsqrt(deg)
    xl = jnp.dot(x_ref[...], w_ref[...], preferred_element_type=jnp.float32)
    y_ref[...] = dinv * xl
    dinv_ref[...] = dinv


def _tc_lin1(x, W1, d0, d1):
    grid = (N // _R,)
    return pl.pallas_call(
        _tc_lin1_body,
        grid=grid,
        in_specs=[
            pl.BlockSpec((_R, D_IN), lambda i: (i, 0)),
            pl.BlockSpec((D_IN, D_H), lambda i: (0, 0)),
            pl.BlockSpec((_R, 1), lambda i: (i, 0)),
            pl.BlockSpec((_R, 1), lambda i: (i, 0)),
        ],
        out_specs=[
            pl.BlockSpec((_R, D_H), lambda i: (i, 0)),
            pl.BlockSpec((_R, 1), lambda i: (i, 0)),
        ],
        out_shape=[
            jax.ShapeDtypeStruct((N, D_H), jnp.float32),
            jax.ShapeDtypeStruct((N, 1), jnp.float32),
        ],
    )(x, W1, d0, d1)


def _tc_mid_body(q0_ref, q1_ref, y_ref, dinv_ref, b_ref, w_ref, y2_ref):
    dinv = dinv_ref[...]
    msg = q0_ref[0] + q1_ref[0] + y_ref[...]
    h = dinv * msg + b_ref[...]
    h = jnp.maximum(h, 0.0)
    y2_ref[...] = dinv * jnp.dot(h, w_ref[...],
                                 preferred_element_type=jnp.float32)


def _tc_mid(q0, q1, y1, dinv, b1, W2):
    grid = (N // _R,)
    return pl.pallas_call(
        _tc_mid_body,
        grid=grid,
        in_specs=[
            pl.BlockSpec((1, _R, D_H), lambda i: (0, i, 0)),
            pl.BlockSpec((1, _R, D_H), lambda i: (1, i, 0)),
            pl.BlockSpec((_R, D_H), lambda i: (i, 0)),
            pl.BlockSpec((_R, 1), lambda i: (i, 0)),
            pl.BlockSpec((1, D_H), lambda i: (0, 0)),
            pl.BlockSpec((D_H, D_H), lambda i: (0, 0)),
        ],
        out_specs=pl.BlockSpec((_R, D_H), lambda i: (i, 0)),
        out_shape=jax.ShapeDtypeStruct((N, D_H), jnp.float32),
    )(q0, q1, y1, dinv, b1, W2)


def _tc_fin_body(r0_ref, r1_ref, y_ref, dinv_ref, b_ref, o_ref):
    msg = r0_ref[0] + r1_ref[0] + y_ref[...]
    h = dinv_ref[...] * msg + b_ref[...]
    o_ref[...] = jnp.maximum(h, 0.0)


def _tc_fin(r0, r1, y2, dinv, b2):
    grid = (N // _R,)
    return pl.pallas_call(
        _tc_fin_body,
        grid=grid,
        in_specs=[
            pl.BlockSpec((1, _R, D_H), lambda i: (0, i, 0)),
            pl.BlockSpec((1, _R, D_H), lambda i: (1, i, 0)),
            pl.BlockSpec((_R, D_H), lambda i: (i, 0)),
            pl.BlockSpec((_R, 1), lambda i: (i, 0)),
            pl.BlockSpec((1, D_H), lambda i: (0, 0)),
        ],
        out_specs=pl.BlockSpec((_R, D_H), lambda i: (i, 0)),
        out_shape=jax.ShapeDtypeStruct((N, D_H), jnp.float32),
    )(r0, r1, y2, dinv, b2)


def kernel(x, edge_index, W1, b1, W2, b2):
    src = edge_index[0]
    dst = edge_index[1]
    npad = E_PAD - E
    pidx = jnp.arange(npad, dtype=jnp.int32)
    src_p = jnp.concatenate([src, pidx % N])
    dst_p = jnp.concatenate([dst, N + pidx % (A - N)])

    zeros_r = jnp.zeros((STRIPE, D_H), jnp.float32)
    zeros_a = jnp.zeros((A,), jnp.float32)

    src3 = jnp.reshape(src_p, (NW, CPW, CH))
    dst3 = jnp.reshape(dst_p, (NW, CPW, CH))

    degp = _sc_degree(dst3, zeros_a)
    d0 = jnp.reshape(degp[0, :N], (N, 1))
    d1 = jnp.reshape(degp[1, :N], (N, 1))
    y1, dinv = _tc_lin1(x, W1, d0, d1)

    q = _sc_edge_pass(y1, src3, dst3, zeros_r)
    y2 = _tc_mid(q, q, y1, dinv, jnp.reshape(b1, (1, D_H)), W2)

    r = _sc_edge_pass(y2, src3, dst3, zeros_r)
    out = _tc_fin(r, r, y2, dinv, jnp.reshape(b2, (1, D_H)))
    return out


# speedup vs baseline: 1.0528x; 1.0528x over previous
import functools

import jax
import jax.numpy as jnp
from jax import lax
from jax.experimental import pallas as pl
from jax.experimental.pallas import tpu as pltpu
from jax.experimental.pallas import tpu_sc as plsc

N = 10000
E = 320000
D_IN = 128
D_H = 64

NC = 2
NS = 16
NW = NC * NS
EPW = E // NW
CH = 128
NFULL = EPW // CH
TAIL = EPW - NFULL * CH
NBUF = 6
A = 10240
STRIPE = A // NS

_MESH = plsc.VectorSubcoreMesh(core_axis_name="c", subcore_axis_name="s")


@functools.partial(
    pl.kernel,
    out_type=jax.ShapeDtypeStruct((NC, A), jnp.float32),
    mesh=_MESH,
    compiler_params=pltpu.CompilerParams(needs_layout_passes=False),
    scratch_types=[
        pltpu.VMEM((EPW,), jnp.int32),
        pltpu.VMEM((A,), jnp.float32),
        pltpu.VMEM((NS, STRIPE), jnp.float32),
        pltpu.VMEM((STRIPE,), jnp.float32),
        pltpu.VMEM_SHARED((NS, A), jnp.float32),
        pltpu.SemaphoreType.DMA,
    ],
)
def _sc_degree(dst_hbm, zeros_hbm, out_hbm, di_all, hist, red, outv, hist_sh, sem):
    c = lax.axis_index("c")
    s = lax.axis_index("s")
    wid = c * NS + s
    pltpu.sync_copy(dst_hbm.at[pl.ds(wid * EPW, EPW)], di_all)
    pltpu.sync_copy(zeros_hbm, hist)
    ones16 = jnp.full((16,), 1.0, jnp.float32)

    def body(j, carry):
        idx = di_all[pl.ds(j * 16, 16)]
        plsc.addupdate_scatter(hist, [idx], ones16)
        return carry

    lax.fori_loop(0, EPW // 16, body, 0)
    pltpu.sync_copy(hist, hist_sh.at[s])
    plsc.subcore_barrier()
    pltpu.sync_copy(hist_sh.at[:, pl.ds(s * STRIPE, STRIPE)], red)

    def rbody(t, carry):
        acc = jnp.zeros((16,), jnp.float32)
        for r in range(NS):
            acc = acc + red[r, pl.ds(t * 16, 16)]
        outv[pl.ds(t * 16, 16)] = acc
        return carry

    lax.fori_loop(0, STRIPE // 16, rbody, 0)
    pltpu.sync_copy(outv, out_hbm.at[c, pl.ds(s * STRIPE, STRIPE)])


@functools.partial(
    pl.kernel,
    out_type=jax.ShapeDtypeStruct((NC, A, D_H), jnp.float32),
    mesh=_MESH,
    compiler_params=pltpu.CompilerParams(use_tc_tiling_on_sc=False),
    scratch_types=(
        [pltpu.VMEM((EPW,), jnp.int32)]
        + [pltpu.VMEM((CH,), jnp.int32) for _ in range(NBUF)]
        + [pltpu.VMEM((TAIL,), jnp.int32)]
        + [pltpu.VMEM((CH, D_H), jnp.float32) for _ in range(NBUF)]
        + [pltpu.VMEM((TAIL, D_H), jnp.float32)]
        + [pltpu.VMEM_SHARED((A, D_H), jnp.float32)]
        + [pltpu.SemaphoreType.DMA for _ in range(2 * NBUF + 1)]
    ),
)
def _sc_edge_pass(y_hbm, src_hbm, dst_hbm, zeros_hbm, out_hbm, *scr):
    si_all = scr[0]
    dibs = scr[1:1 + NBUF]
    di_t = scr[1 + NBUF]
    rows_bufs = scr[2 + NBUF:2 + 2 * NBUF]
    rows_t = scr[2 + 2 * NBUF]
    acc_sh = scr[3 + 2 * NBUF]
    gsems = scr[4 + 2 * NBUF:4 + 3 * NBUF]
    dsems = scr[4 + 3 * NBUF:4 + 4 * NBUF]
    ssem = scr[4 + 4 * NBUF]
    c = lax.axis_index("c")
    s = lax.axis_index("s")
    wid = c * NS + s
    ebase = wid * EPW
    pltpu.sync_copy(src_hbm.at[pl.ds(ebase, EPW)], si_all)
    pltpu.sync_copy(zeros_hbm, acc_sh.at[pl.ds(s * STRIPE, STRIPE)])
    plsc.subcore_barrier()

    def body(m, carry):
        gds, dds, sds = [], [], []
        for b in range(NBUF):
            j = m * NBUF + b
            dds.append(pltpu.async_copy(
                dst_hbm.at[pl.ds(ebase + j * CH, CH)], dibs[b], dsems[b]))
            gds.append(pltpu.async_copy(
                y_hbm.at[si_all.at[pl.ds(j * CH, CH)]], rows_bufs[b],
                gsems[b]))
        for b in range(NBUF):
            gds[b].wait()
            dds[b].wait()
            sds.append(pltpu.async_copy(
                rows_bufs[b], acc_sh.at[dibs[b]], ssem, add=True))
        for b in range(NBUF):
            sds[b].wait()
        return carry

    lax.fori_loop(0, NFULL // NBUF, body, 0)
    pltpu.sync_copy(dst_hbm.at[pl.ds(ebase + NFULL * CH, TAIL)], di_t)
    pltpu.async_copy(
        y_hbm.at[si_all.at[pl.ds(NFULL * CH, TAIL)]], rows_t, gsems[0]
    ).wait()
    pltpu.sync_copy(rows_t, acc_sh.at[di_t], add=True)

    plsc.subcore_barrier()
    pltpu.sync_copy(acc_sh.at[pl.ds(s * STRIPE, STRIPE)],
                    out_hbm.at[c, pl.ds(s * STRIPE, STRIPE)])


_R = 2000


def _tc_lin1_body(x_ref, w_ref, d0_ref, d1_ref, y_ref, dinv_ref):
    deg = d0_ref[...] + d1_ref[...] + 1.0
    dinv = lax.rsqrt(deg)
    xl = jnp.dot(x_ref[...], w_ref[...], preferred_element_type=jnp.float32)
    y_ref[...] = dinv * xl
    dinv_ref[...] = dinv


def _tc_lin1(x, W1, d0, d1):
    grid = (N // _R,)
    return pl.pallas_call(
        _tc_lin1_body,
        grid=grid,
        in_specs=[
            pl.BlockSpec((_R, D_IN), lambda i: (i, 0)),
            pl.BlockSpec((D_IN, D_H), lambda i: (0, 0)),
            pl.BlockSpec((_R, 1), lambda i: (i, 0)),
            pl.BlockSpec((_R, 1), lambda i: (i, 0)),
        ],
        out_specs=[
            pl.BlockSpec((_R, D_H), lambda i: (i, 0)),
            pl.BlockSpec((_R, 1), lambda i: (i, 0)),
        ],
        out_shape=[
            jax.ShapeDtypeStruct((N, D_H), jnp.float32),
            jax.ShapeDtypeStruct((N, 1), jnp.float32),
        ],
    )(x, W1, d0, d1)


def _tc_mid_body(q0_ref, q1_ref, y_ref, dinv_ref, b_ref, w_ref, y2_ref):
    dinv = dinv_ref[...]
    msg = q0_ref[0] + q1_ref[0] + y_ref[...]
    h = dinv * msg + b_ref[...]
    h = jnp.maximum(h, 0.0)
    y2_ref[...] = dinv * jnp.dot(h, w_ref[...],
                                 preferred_element_type=jnp.float32)


def _tc_mid(q, y1, dinv, b1, W2):
    grid = (N // _R,)
    return pl.pallas_call(
        _tc_mid_body,
        grid=grid,
        in_specs=[
            pl.BlockSpec((1, _R, D_H), lambda i: (0, i, 0)),
            pl.BlockSpec((1, _R, D_H), lambda i: (1, i, 0)),
            pl.BlockSpec((_R, D_H), lambda i: (i, 0)),
            pl.BlockSpec((_R, 1), lambda i: (i, 0)),
            pl.BlockSpec((1, D_H), lambda i: (0, 0)),
            pl.BlockSpec((D_H, D_H), lambda i: (0, 0)),
        ],
        out_specs=pl.BlockSpec((_R, D_H), lambda i: (i, 0)),
        out_shape=jax.ShapeDtypeStruct((N, D_H), jnp.float32),
    )(q, q, y1, dinv, b1, W2)


def _tc_fin_body(r0_ref, r1_ref, y_ref, dinv_ref, b_ref, o_ref):
    msg = r0_ref[0] + r1_ref[0] + y_ref[...]
    h = dinv_ref[...] * msg + b_ref[...]
    o_ref[...] = jnp.maximum(h, 0.0)


def _tc_fin(r, y2, dinv, b2):
    grid = (N // _R,)
    return pl.pallas_call(
        _tc_fin_body,
        grid=grid,
        in_specs=[
            pl.BlockSpec((1, _R, D_H), lambda i: (0, i, 0)),
            pl.BlockSpec((1, _R, D_H), lambda i: (1, i, 0)),
            pl.BlockSpec((_R, D_H), lambda i: (i, 0)),
            pl.BlockSpec((_R, 1), lambda i: (i, 0)),
            pl.BlockSpec((1, D_H), lambda i: (0, 0)),
        ],
        out_specs=pl.BlockSpec((_R, D_H), lambda i: (i, 0)),
        out_shape=jax.ShapeDtypeStruct((N, D_H), jnp.float32),
    )(r, r, y2, dinv, b2)


def kernel(x, edge_index, W1, b1, W2, b2):
    src = edge_index[0]
    dst = edge_index[1]

    zeros_r = jnp.zeros((STRIPE, D_H), jnp.float32)
    zeros_a = jnp.zeros((A,), jnp.float32)

    degp = _sc_degree(dst, zeros_a)
    d0 = jnp.reshape(degp[0, :N], (N, 1))
    d1 = jnp.reshape(degp[1, :N], (N, 1))
    y1, dinv = _tc_lin1(x, W1, d0, d1)

    q = _sc_edge_pass(y1, src, dst, zeros_r)
    y2 = _tc_mid(q, y1, dinv, jnp.reshape(b1, (1, D_H)), W2)

    r = _sc_edge_pass(y2, src, dst, zeros_r)
    out = _tc_fin(r, y2, dinv, jnp.reshape(b2, (1, D_H)))
    return out


# speedup vs baseline: 1.0967x; 1.0417x over previous
import functools

import jax
import jax.numpy as jnp
from jax import lax
from jax.experimental import pallas as pl
from jax.experimental.pallas import tpu as pltpu
from jax.experimental.pallas import tpu_sc as plsc

N = 10000
E = 320000
D_IN = 128
D_H = 64

NC = 2
NS = 16
NW = NC * NS
CH = 128
NCHUNK = E // CH
NFULL = NCHUNK // NW
NEXTRA = NCHUNK - NFULL * NW
EPW = NFULL * CH
NBUF = 6
A = 10240
STRIPE = A // NS

_MESH = plsc.VectorSubcoreMesh(core_axis_name="c", subcore_axis_name="s")


@functools.partial(
    pl.kernel,
    out_type=jax.ShapeDtypeStruct((NC, A), jnp.float32),
    mesh=_MESH,
    compiler_params=pltpu.CompilerParams(
        needs_layout_passes=False, use_tc_tiling_on_sc=False),
    scratch_types=[
        pltpu.VMEM((EPW,), jnp.int32),
        pltpu.VMEM((CH,), jnp.int32),
        pltpu.VMEM((A,), jnp.float32),
        pltpu.VMEM((NS, STRIPE), jnp.float32),
        pltpu.VMEM((STRIPE,), jnp.float32),
        pltpu.VMEM_SHARED((NS, A), jnp.float32),
        pltpu.SemaphoreType.DMA,
    ],
)
def _sc_degree(ei_hbm, zeros_hbm, out_hbm, di_all, di_x, hist, red, outv,
               hist_sh, sem):
    c = lax.axis_index("c")
    s = lax.axis_index("s")
    wid = c * NS + s
    pltpu.sync_copy(ei_hbm.at[1, pl.ds(wid * EPW, EPW)], di_all)
    pltpu.sync_copy(zeros_hbm, hist)
    ones16 = jnp.full((16,), 1.0, jnp.float32)

    def body(j, carry):
        idx = di_all[pl.ds(j * 16, 16)]
        plsc.addupdate_scatter(hist, [idx], ones16)
        return carry

    lax.fori_loop(0, EPW // 16, body, 0)

    @pl.when(wid < NEXTRA)
    def _():
        pltpu.sync_copy(
            ei_hbm.at[1, pl.ds((NW * NFULL + wid) * CH, CH)], di_x)

        def xbody(j, carry):
            idx = di_x[pl.ds(j * 16, 16)]
            plsc.addupdate_scatter(hist, [idx], ones16)
            return carry

        lax.fori_loop(0, CH // 16, xbody, 0)
    pltpu.sync_copy(hist, hist_sh.at[s])
    plsc.subcore_barrier()
    pltpu.sync_copy(hist_sh.at[:, pl.ds(s * STRIPE, STRIPE)], red)

    def rbody(t, carry):
        acc = jnp.zeros((16,), jnp.float32)
        for r in range(NS):
            acc = acc + red[r, pl.ds(t * 16, 16)]
        outv[pl.ds(t * 16, 16)] = acc
        return carry

    lax.fori_loop(0, STRIPE // 16, rbody, 0)
    pltpu.sync_copy(outv, out_hbm.at[c, pl.ds(s * STRIPE, STRIPE)])


@functools.partial(
    pl.kernel,
    out_type=jax.ShapeDtypeStruct((NC, A, D_H), jnp.float32),
    mesh=_MESH,
    compiler_params=pltpu.CompilerParams(use_tc_tiling_on_sc=False),
    scratch_types=(
        [pltpu.VMEM((EPW,), jnp.int32)]
        + [pltpu.VMEM((CH,), jnp.int32) for _ in range(NBUF)]
        + [pltpu.VMEM((CH,), jnp.int32)]
        + [pltpu.VMEM((CH, D_H), jnp.float32) for _ in range(NBUF)]
        + [pltpu.VMEM_SHARED((A, D_H), jnp.float32)]
        + [pltpu.SemaphoreType.DMA for _ in range(2 * NBUF + 1)]
    ),
)
def _sc_edge_pass(y_hbm, ei_hbm, zeros_hbm, out_hbm, *scr):
    si_all = scr[0]
    dibs = scr[1:1 + NBUF]
    si_x = scr[1 + NBUF]
    rows_bufs = scr[2 + NBUF:2 + 2 * NBUF]
    acc_sh = scr[2 + 2 * NBUF]
    gsems = scr[3 + 2 * NBUF:3 + 3 * NBUF]
    dsems = scr[3 + 3 * NBUF:3 + 4 * NBUF]
    ssem = scr[3 + 4 * NBUF]
    c = lax.axis_index("c")
    s = lax.axis_index("s")
    wid = c * NS + s
    ebase = wid * EPW
    pltpu.sync_copy(ei_hbm.at[0, pl.ds(ebase, EPW)], si_all)
    pltpu.sync_copy(zeros_hbm, acc_sh.at[pl.ds(s * STRIPE, STRIPE)])
    plsc.subcore_barrier()

    def body(m, carry):
        gds, dds, sds = [], [], []
        for b in range(NBUF):
            j = m * NBUF + b
            dds.append(pltpu.async_copy(
                ei_hbm.at[1, pl.ds(ebase + j * CH, CH)], dibs[b], dsems[b]))
            gds.append(pltpu.async_copy(
                y_hbm.at[si_all.at[pl.ds(j * CH, CH)]], rows_bufs[b],
                gsems[b]))
        for b in range(NBUF):
            gds[b].wait()
            dds[b].wait()
            sds.append(pltpu.async_copy(
                rows_bufs[b], acc_sh.at[dibs[b]], ssem, add=True))
        for b in range(NBUF):
            sds[b].wait()
        return carry

    lax.fori_loop(0, NFULL // NBUF, body, 0)

    @pl.when(wid < NEXTRA)
    def _():
        xoff = (NW * NFULL + wid) * CH
        pltpu.sync_copy(ei_hbm.at[0, pl.ds(xoff, CH)], si_x)
        pltpu.sync_copy(ei_hbm.at[1, pl.ds(xoff, CH)], dibs[0])
        pltpu.async_copy(y_hbm.at[si_x], rows_bufs[0], gsems[0]).wait()
        pltpu.sync_copy(rows_bufs[0], acc_sh.at[dibs[0]], add=True)

    plsc.subcore_barrier()
    pltpu.sync_copy(acc_sh.at[pl.ds(s * STRIPE, STRIPE)],
                    out_hbm.at[c, pl.ds(s * STRIPE, STRIPE)])


_R = 2000


def _tc_lin1_body(x_ref, w_ref, d0_ref, d1_ref, y_ref, dinv_ref):
    deg = d0_ref[...] + d1_ref[...] + 1.0
    dinv = lax.rsqrt(deg)
    xl = jnp.dot(x_ref[...], w_ref[...], preferred_element_type=jnp.float32)
    y_ref[...] = dinv * xl
    dinv_ref[...] = dinv


def _tc_lin1(x, W1, d0, d1):
    grid = (N // _R,)
    return pl.pallas_call(
        _tc_lin1_body,
        grid=grid,
        in_specs=[
            pl.BlockSpec((_R, D_IN), lambda i: (i, 0)),
            pl.BlockSpec((D_IN, D_H), lambda i: (0, 0)),
            pl.BlockSpec((_R, 1), lambda i: (i, 0)),
            pl.BlockSpec((_R, 1), lambda i: (i, 0)),
        ],
        out_specs=[
            pl.BlockSpec((_R, D_H), lambda i: (i, 0)),
            pl.BlockSpec((_R, 1), lambda i: (i, 0)),
        ],
        out_shape=[
            jax.ShapeDtypeStruct((N, D_H), jnp.float32),
            jax.ShapeDtypeStruct((N, 1), jnp.float32),
        ],
    )(x, W1, d0, d1)


def _tc_mid_body(q0_ref, q1_ref, y_ref, dinv_ref, b_ref, w_ref, y2_ref):
    dinv = dinv_ref[...]
    msg = q0_ref[0] + q1_ref[0] + y_ref[...]
    h = dinv * msg + b_ref[...]
    h = jnp.maximum(h, 0.0)
    y2_ref[...] = dinv * jnp.dot(h, w_ref[...],
                                 preferred_element_type=jnp.float32)


def _tc_mid(q, y1, dinv, b1, W2):
    grid = (N // _R,)
    return pl.pallas_call(
        _tc_mid_body,
        grid=grid,
        in_specs=[
            pl.BlockSpec((1, _R, D_H), lambda i: (0, i, 0)),
            pl.BlockSpec((1, _R, D_H), lambda i: (1, i, 0)),
            pl.BlockSpec((_R, D_H), lambda i: (i, 0)),
            pl.BlockSpec((_R, 1), lambda i: (i, 0)),
            pl.BlockSpec((1, D_H), lambda i: (0, 0)),
            pl.BlockSpec((D_H, D_H), lambda i: (0, 0)),
        ],
        out_specs=pl.BlockSpec((_R, D_H), lambda i: (i, 0)),
        out_shape=jax.ShapeDtypeStruct((N, D_H), jnp.float32),
    )(q, q, y1, dinv, b1, W2)


def _tc_fin_body(r0_ref, r1_ref, y_ref, dinv_ref, b_ref, o_ref):
    msg = r0_ref[0] + r1_ref[0] + y_ref[...]
    h = dinv_ref[...] * msg + b_ref[...]
    o_ref[...] = jnp.maximum(h, 0.0)


def _tc_fin(r, y2, dinv, b2):
    grid = (N // _R,)
    return pl.pallas_call(
        _tc_fin_body,
        grid=grid,
        in_specs=[
            pl.BlockSpec((1, _R, D_H), lambda i: (0, i, 0)),
            pl.BlockSpec((1, _R, D_H), lambda i: (1, i, 0)),
            pl.BlockSpec((_R, D_H), lambda i: (i, 0)),
            pl.BlockSpec((_R, 1), lambda i: (i, 0)),
            pl.BlockSpec((1, D_H), lambda i: (0, 0)),
        ],
        out_specs=pl.BlockSpec((_R, D_H), lambda i: (i, 0)),
        out_shape=jax.ShapeDtypeStruct((N, D_H), jnp.float32),
    )(r, r, y2, dinv, b2)


def kernel(x, edge_index, W1, b1, W2, b2):
    zeros_r = jnp.zeros((STRIPE, D_H), jnp.float32)
    zeros_a = jnp.zeros((A,), jnp.float32)

    degp = _sc_degree(edge_index, zeros_a)
    d0 = jnp.reshape(degp[0, :N], (N, 1))
    d1 = jnp.reshape(degp[1, :N], (N, 1))
    y1, dinv = _tc_lin1(x, W1, d0, d1)

    q = _sc_edge_pass(y1, edge_index, zeros_r)
    y2 = _tc_mid(q, y1, dinv, jnp.reshape(b1, (1, D_H)), W2)

    r = _sc_edge_pass(y2, edge_index, zeros_r)
    out = _tc_fin(r, y2, dinv, jnp.reshape(b2, (1, D_H)))
    return out


# speedup vs baseline: 1.2159x; 1.1086x over previous
import functools

import jax
import jax.numpy as jnp
from jax import lax
from jax.experimental import pallas as pl
from jax.experimental.pallas import tpu as pltpu
from jax.experimental.pallas import tpu_sc as plsc

N = 10000
E = 320000
D_IN = 128
D_H = 64

NC = 2
NS = 16
NW = NC * NS
CH = 128
NCHUNK = E // CH
NFULL = NCHUNK // NW
NEXTRA = NCHUNK - NFULL * NW
EPW = NFULL * CH
NBUF = 6
A = 10240
STRIPE = A // NS

_MESH = plsc.VectorSubcoreMesh(core_axis_name="c", subcore_axis_name="s")


@functools.partial(
    pl.kernel,
    out_type=jax.ShapeDtypeStruct((NC, A), jnp.float32),
    mesh=_MESH,
    compiler_params=pltpu.CompilerParams(
        needs_layout_passes=False, use_tc_tiling_on_sc=False),
    scratch_types=[
        pltpu.VMEM((EPW,), jnp.int32),
        pltpu.VMEM((CH,), jnp.int32),
        pltpu.VMEM((A,), jnp.float32),
        pltpu.VMEM((NS, STRIPE), jnp.float32),
        pltpu.VMEM((STRIPE,), jnp.float32),
        pltpu.VMEM_SHARED((NS, A), jnp.float32),
        pltpu.SemaphoreType.DMA,
    ],
)
def _sc_degree(ei_hbm, zeros_hbm, out_hbm, di_all, di_x, hist, red, outv,
               hist_sh, sem):
    c = lax.axis_index("c")
    s = lax.axis_index("s")
    wid = c * NS + s
    pltpu.sync_copy(ei_hbm.at[1, pl.ds(wid * EPW, EPW)], di_all)
    pltpu.sync_copy(zeros_hbm, hist)
    ones16 = jnp.full((16,), 1.0, jnp.float32)

    def body(j, carry):
        idx = di_all[pl.ds(j * 16, 16)]
        plsc.addupdate_scatter(hist, [idx], ones16)
        return carry

    lax.fori_loop(0, EPW // 16, body, 0)

    @pl.when(wid < NEXTRA)
    def _():
        pltpu.sync_copy(
            ei_hbm.at[1, pl.ds((NW * NFULL + wid) * CH, CH)], di_x)

        def xbody(j, carry):
            idx = di_x[pl.ds(j * 16, 16)]
            plsc.addupdate_scatter(hist, [idx], ones16)
            return carry

        lax.fori_loop(0, CH // 16, xbody, 0)
    pltpu.sync_copy(hist, hist_sh.at[s])
    plsc.subcore_barrier()
    pltpu.sync_copy(hist_sh.at[:, pl.ds(s * STRIPE, STRIPE)], red)

    def rbody(t, carry):
        acc = jnp.zeros((16,), jnp.float32)
        for r in range(NS):
            acc = acc + red[r, pl.ds(t * 16, 16)]
        outv[pl.ds(t * 16, 16)] = acc
        return carry

    lax.fori_loop(0, STRIPE // 16, rbody, 0)
    pltpu.sync_copy(outv, out_hbm.at[c, pl.ds(s * STRIPE, STRIPE)])


@functools.partial(
    pl.kernel,
    out_type=jax.ShapeDtypeStruct((NC, A, 2 * D_H), jnp.float32),
    mesh=_MESH,
    compiler_params=pltpu.CompilerParams(use_tc_tiling_on_sc=False),
    scratch_types=(
        [pltpu.VMEM((EPW,), jnp.int32)]
        + [pltpu.VMEM((CH,), jnp.int32) for _ in range(NBUF)]
        + [pltpu.VMEM((CH,), jnp.int32)]
        + [pltpu.VMEM((CH, D_H), jnp.float32) for _ in range(NBUF)]
        + [pltpu.VMEM_SHARED((A, D_H), jnp.float32)]
        + [pltpu.SemaphoreType.DMA for _ in range(2 * NBUF + 1)]
    ),
)
def _sc_edge_pass(y_hbm, ei_hbm, zeros_hbm, out_hbm, *scr):
    si_all = scr[0]
    dibs = scr[1:1 + NBUF]
    si_x = scr[1 + NBUF]
    rows_bufs = scr[2 + NBUF:2 + 2 * NBUF]
    acc_sh = scr[2 + 2 * NBUF]
    gsems = scr[3 + 2 * NBUF:3 + 3 * NBUF]
    dsems = scr[3 + 3 * NBUF:3 + 4 * NBUF]
    ssem = scr[3 + 4 * NBUF]
    c = lax.axis_index("c")
    s = lax.axis_index("s")
    wid = c * NS + s
    ebase = wid * EPW
    pltpu.sync_copy(ei_hbm.at[0, pl.ds(ebase, EPW)], si_all)
    pltpu.sync_copy(zeros_hbm, acc_sh.at[pl.ds(s * STRIPE, STRIPE)])

    def dbl(k, carry):
        v = si_all[pl.ds(k * 16, 16)]
        si_all[pl.ds(k * 16, 16)] = v + v
        return carry

    lax.fori_loop(0, EPW // 16, dbl, 0)
    plsc.subcore_barrier()

    def body(m, carry):
        gds, dds, sds = [], [], []
        for b in range(NBUF):
            j = m * NBUF + b
            dds.append(pltpu.async_copy(
                ei_hbm.at[1, pl.ds(ebase + j * CH, CH)], dibs[b], dsems[b]))
            gds.append(pltpu.async_copy(
                y_hbm.at[si_all.at[pl.ds(j * CH, CH)]], rows_bufs[b],
                gsems[b]))
        for b in range(NBUF):
            gds[b].wait()
            dds[b].wait()
            sds.append(pltpu.async_copy(
                rows_bufs[b], acc_sh.at[dibs[b]], ssem, add=True))
        for b in range(NBUF):
            sds[b].wait()
        return carry

    lax.fori_loop(0, NFULL // NBUF, body, 0)

    @pl.when(wid < NEXTRA)
    def _():
        xoff = (NW * NFULL + wid) * CH
        pltpu.sync_copy(ei_hbm.at[0, pl.ds(xoff, CH)], si_x)

        def dblx(k, carry):
            v = si_x[pl.ds(k * 16, 16)]
            si_x[pl.ds(k * 16, 16)] = v + v
            return carry

        lax.fori_loop(0, CH // 16, dblx, 0)
        pltpu.sync_copy(ei_hbm.at[1, pl.ds(xoff, CH)], dibs[0])
        pltpu.async_copy(y_hbm.at[si_x], rows_bufs[0], gsems[0]).wait()
        pltpu.sync_copy(rows_bufs[0], acc_sh.at[dibs[0]], add=True)

    plsc.subcore_barrier()
    pltpu.sync_copy(acc_sh.at[pl.ds(s * STRIPE, STRIPE)],
                    out_hbm.at[c, pl.ds(s * STRIPE, STRIPE), pl.ds(0, D_H)])


_R = 2000


def _tc_lin1_body(x_ref, w_ref, d0_ref, d1_ref, y_ref, dinv_ref):
    deg = d0_ref[...] + d1_ref[...] + 1.0
    dinv = lax.rsqrt(deg)
    xl = jnp.dot(x_ref[...], w_ref[...], preferred_element_type=jnp.float32)
    y_ref[...] = jnp.concatenate(
        [dinv * xl, jnp.zeros((_R, D_H), jnp.float32)], axis=1)
    dinv_ref[...] = dinv


def _tc_lin1(x, W1, d0, d1):
    grid = (N // _R,)
    return pl.pallas_call(
        _tc_lin1_body,
        grid=grid,
        in_specs=[
            pl.BlockSpec((_R, D_IN), lambda i: (i, 0)),
            pl.BlockSpec((D_IN, D_H), lambda i: (0, 0)),
            pl.BlockSpec((_R, 1), lambda i: (i, 0)),
            pl.BlockSpec((_R, 1), lambda i: (i, 0)),
        ],
        out_specs=[
            pl.BlockSpec((_R, 2 * D_H), lambda i: (i, 0)),
            pl.BlockSpec((_R, 1), lambda i: (i, 0)),
        ],
        out_shape=[
            jax.ShapeDtypeStruct((N, 2 * D_H), jnp.float32),
            jax.ShapeDtypeStruct((N, 1), jnp.float32),
        ],
    )(x, W1, d0, d1)


def _tc_mid_body(q0_ref, q1_ref, y_ref, dinv_ref, b_ref, w_ref, y2_ref):
    dinv = dinv_ref[...]
    msg = (q0_ref[0] + q1_ref[0])[:, :D_H] + y_ref[:, :D_H]
    h = dinv * msg + b_ref[...]
    h = jnp.maximum(h, 0.0)
    y2 = dinv * jnp.dot(h, w_ref[...], preferred_element_type=jnp.float32)
    y2_ref[...] = jnp.concatenate(
        [y2, jnp.zeros((_R, D_H), jnp.float32)], axis=1)


def _tc_mid(q, y1, dinv, b1, W2):
    grid = (N // _R,)
    return pl.pallas_call(
        _tc_mid_body,
        grid=grid,
        in_specs=[
            pl.BlockSpec((1, _R, 2 * D_H), lambda i: (0, i, 0)),
            pl.BlockSpec((1, _R, 2 * D_H), lambda i: (1, i, 0)),
            pl.BlockSpec((_R, 2 * D_H), lambda i: (i, 0)),
            pl.BlockSpec((_R, 1), lambda i: (i, 0)),
            pl.BlockSpec((1, D_H), lambda i: (0, 0)),
            pl.BlockSpec((D_H, D_H), lambda i: (0, 0)),
        ],
        out_specs=pl.BlockSpec((_R, 2 * D_H), lambda i: (i, 0)),
        out_shape=jax.ShapeDtypeStruct((N, 2 * D_H), jnp.float32),
    )(q, q, y1, dinv, b1, W2)


def _tc_fin_body(r0_ref, r1_ref, y_ref, dinv_ref, b_ref, o_ref):
    msg = (r0_ref[0] + r1_ref[0])[:, :D_H] + y_ref[:, :D_H]
    h = dinv_ref[...] * msg + b_ref[...]
    o_ref[...] = jnp.maximum(h, 0.0)


def _tc_fin(r, y2, dinv, b2):
    grid = (N // _R,)
    return pl.pallas_call(
        _tc_fin_body,
        grid=grid,
        in_specs=[
            pl.BlockSpec((1, _R, 2 * D_H), lambda i: (0, i, 0)),
            pl.BlockSpec((1, _R, 2 * D_H), lambda i: (1, i, 0)),
            pl.BlockSpec((_R, 2 * D_H), lambda i: (i, 0)),
            pl.BlockSpec((_R, 1), lambda i: (i, 0)),
            pl.BlockSpec((1, D_H), lambda i: (0, 0)),
        ],
        out_specs=pl.BlockSpec((_R, D_H), lambda i: (i, 0)),
        out_shape=jax.ShapeDtypeStruct((N, D_H), jnp.float32),
    )(r, r, y2, dinv, b2)


def kernel(x, edge_index, W1, b1, W2, b2):
    zeros_r = jnp.zeros((STRIPE, D_H), jnp.float32)
    zeros_a = jnp.zeros((A,), jnp.float32)

    degp = _sc_degree(edge_index, zeros_a)
    d0 = jnp.reshape(degp[0, :N], (N, 1))
    d1 = jnp.reshape(degp[1, :N], (N, 1))
    y1, dinv = _tc_lin1(x, W1, d0, d1)

    q = _sc_edge_pass(jnp.reshape(y1, (2 * N, D_H)), edge_index, zeros_r)
    y2 = _tc_mid(q, y1, dinv, jnp.reshape(b1, (1, D_H)), W2)

    r = _sc_edge_pass(jnp.reshape(y2, (2 * N, D_H)), edge_index, zeros_r)
    out = _tc_fin(r, y2, dinv, jnp.reshape(b2, (1, D_H)))
    return out
